# baseline (speedup 1.0000x reference)
import functools

import jax
import jax.numpy as jnp
from jax import lax
from jax.experimental import pallas as pl
from jax.experimental.pallas import tpu as pltpu
from jax.experimental.pallas import tpu_sc as plsc

DIM = 32
WIDE = 128
ROWS_PER_WIDE = WIDE // DIM
LANES = 16
NUM_CORES = 2
NUM_SUBCORES = 16
NW = NUM_CORES * NUM_SUBCORES
CHUNK = 128


def kernel(users, items, user_emb, item_emb):
    batch = users.shape[0]
    b_per_w = batch // NW
    n_chunks = b_per_w // CHUNK
    mesh = plsc.VectorSubcoreMesh(core_axis_name="c", subcore_axis_name="s")
    cp = pltpu.CompilerParams(needs_layout_passes=False)

    @functools.partial(
        pl.kernel,
        compiler_params=cp,
        out_type=jax.ShapeDtypeStruct((batch,), jnp.float32),
        mesh=mesh,
        scratch_types=[
            pltpu.VMEM((b_per_w,), jnp.int32),
            pltpu.VMEM((b_per_w,), jnp.int32),
            pltpu.VMEM((b_per_w,), jnp.int32),
            pltpu.VMEM((b_per_w,), jnp.int32),
            pltpu.VMEM((CHUNK, WIDE), jnp.float32),
            pltpu.VMEM((CHUNK, WIDE), jnp.float32),
            pltpu.VMEM((b_per_w,), jnp.float32),
            pltpu.SemaphoreType.DMA,
            pltpu.SemaphoreType.DMA,
        ],
    )
    def sc_kernel(users_hbm, items_hbm, uemb_hbm, vemb_hbm, out_hbm,
                  uidx_v, iidx_v, uq_v, iq_v, ug_v, vg_v, out_v,
                  sem_u, sem_v):
        wid = lax.axis_index("s") * NUM_CORES + lax.axis_index("c")
        base = wid * b_per_w
        pltpu.sync_copy(users_hbm.at[pl.ds(base, b_per_w)], uidx_v)
        pltpu.sync_copy(items_hbm.at[pl.ds(base, b_per_w)], iidx_v)

        @pl.loop(0, b_per_w // LANES)
        def _(s):
            j = s * LANES
            uq_v[pl.ds(j, LANES)] = lax.shift_right_logical(
                uidx_v[pl.ds(j, LANES)], 2)
            iq_v[pl.ds(j, LANES)] = lax.shift_right_logical(
                iidx_v[pl.ds(j, LANES)], 2)

        lane = lax.iota(jnp.int32, LANES)

        @pl.loop(0, n_chunks)
        def _(c):
            c0 = c * CHUNK
            cu = pltpu.async_copy(
                uemb_hbm.at[uq_v.at[pl.ds(c0, CHUNK)]], ug_v, sem_u)
            cv = pltpu.async_copy(
                vemb_hbm.at[iq_v.at[pl.ds(c0, CHUNK)]], vg_v, sem_v)
            cu.wait()
            cv.wait()
            for g in range(CHUNK // LANES):
                j0 = c0 + g * LANES
                rows = lane + g * LANES
                ucol = (uidx_v[pl.ds(j0, LANES)] & 3) * DIM
                icol = (iidx_v[pl.ds(j0, LANES)] & 3) * DIM
                acc = None
                for k in range(DIM):
                    u = plsc.load_gather(ug_v, [rows, ucol + k])
                    v = plsc.load_gather(vg_v, [rows, icol + k])
                    acc = u * v if acc is None else acc + u * v
                out_v[pl.ds(j0, LANES)] = acc

        pltpu.sync_copy(out_v, out_hbm.at[pl.ds(base, b_per_w)])

    return sc_kernel(users, items, user_emb.reshape(-1, WIDE),
                     item_emb.reshape(-1, WIDE))
